# baseline (speedup 1.0000x reference)
import functools

import jax
import jax.numpy as jnp
from jax import lax
from jax.experimental import pallas as pl
from jax.experimental.pallas import tpu as pltpu
from jax.experimental.pallas import tpu_sc as plsc

B, LW, LC = 64, 512, 2048
DW, DC, H = 300, 64, 128
DWP = 384
DCP = 128

NC, NS = 2, 16
NW = NC * NS

ROWS_PER_W = B * LW // NW
WCHUNK = 128
NWCHUNK = ROWS_PER_W // WCHUNK
NPIECE = DWP // 128
CCHUNK = 128
ZROWS = LW // NS

_f32 = jnp.float32
_bf16 = jnp.bfloat16
_i32 = jnp.int32

_MESH = plsc.VectorSubcoreMesh(core_axis_name="c", subcore_axis_name="s")


def _scc_body(c_hbm, p_hbm, ctab_hbm, acc_hbm,
              cbuf, pbuf, gbuf, zbuf, acc, sem):
  sid = lax.axis_index("s")
  cid = lax.axis_index("c")

  zseg = jnp.zeros((16,), _f32)

  def _zero_row(i, _):
    for q in range(DCP // 16):
      zbuf[i, pl.ds(q * 16, 16)] = zseg
    return 0

  lax.fori_loop(0, ZROWS, _zero_row, 0)

  myrows = pl.ds(sid * ZROWS, ZROWS)

  def _row(j, _):
    b = cid * (B // NC) + j
    pltpu.sync_copy(c_hbm.at[b, sid], cbuf)
    pltpu.sync_copy(p_hbm.at[b, sid], pbuf)

    for q in range(CCHUNK // 16):
      v = cbuf[pl.ds(q * 16, 16)]
      v = jnp.where(v == 32, 0, v)
      v = jnp.minimum(v, 256)
      cbuf[pl.ds(q * 16, 16)] = v

    pltpu.sync_copy(zbuf, acc.at[myrows])
    plsc.subcore_barrier()
    pltpu.async_copy(ctab_hbm.at[cbuf], gbuf, sem).wait()
    pltpu.sync_copy(gbuf, acc.at[pbuf], add=True)
    plsc.subcore_barrier()
    pltpu.sync_copy(acc.at[myrows], acc_hbm.at[b, myrows])
    plsc.subcore_barrier()
    return 0

  lax.fori_loop(0, B // NC, _row, 0)


_scc_call = functools.partial(
    pl.kernel,
    out_type=jax.ShapeDtypeStruct((B, LW, DCP), _f32),
    mesh=_MESH,
    scratch_types=[
        pltpu.VMEM((CCHUNK,), _i32),
        pltpu.VMEM((CCHUNK,), _i32),
        pltpu.VMEM((CCHUNK, DCP), _f32),
        pltpu.VMEM((ZROWS, DCP), _f32),
        pltpu.VMEM_SHARED((LW, DCP), _f32),
        pltpu.SemaphoreType.DMA,
    ],
)(_scc_body)


def _scw_body(w_hbm, table_hbm, dep_hbm,
              e0_hbm, e1_hbm, e2_hbm,
              widx, b0a, b1a, b2a, b0b, b1b, b2b, gsem, ssem):
  del dep_hbm
  sid = lax.axis_index("s")
  wid = sid * NC + lax.axis_index("c")

  pltpu.sync_copy(w_hbm.at[wid], widx)
  wbase = wid * ROWS_PER_W
  srcs = tuple(table_hbm.at[:, pl.ds(128 * i, 128)] for i in range(NPIECE))
  outs = (e0_hbm, e1_hbm, e2_hbm)
  bufsets = ((b0a, b1a, b2a), (b0b, b1b, b2b))

  del bufsets, ssem
  for k in range(NWCHUNK):
    rows = pl.ds(wbase + k * WCHUNK, WCHUNK)
    hs = [pltpu.async_copy(srcs[i].at[widx.at[k]], (b0a, b1a, b2a)[i], gsem)
          for i in range(NPIECE)]
    for h in hs:
      h.wait()
    for i in range(NPIECE):
      pltpu.sync_copy((b0a, b1a, b2a)[i], outs[i].at[rows])


_scw_call = functools.partial(
    pl.kernel,
    out_type=(
        jax.ShapeDtypeStruct((B * LW, 128), _f32),
        jax.ShapeDtypeStruct((B * LW, 128), _f32),
        jax.ShapeDtypeStruct((B * LW, 128), _f32),
    ),
    mesh=_MESH,
    scratch_types=[
        pltpu.VMEM((NWCHUNK, WCHUNK), _i32),
        pltpu.VMEM((WCHUNK, 128), _f32),
        pltpu.VMEM((WCHUNK, 128), _f32),
        pltpu.VMEM((WCHUNK, 128), _f32),
        pltpu.VMEM((WCHUNK, 128), _f32),
        pltpu.VMEM((WCHUNK, 128), _f32),
        pltpu.VMEM((WCHUNK, 128), _f32),
        pltpu.SemaphoreType.DMA,
        pltpu.SemaphoreType.DMA,
    ],
)(_scw_body)


def _tc_body(e0_ref, e1_ref, e2_ref, acc_ref, wp0_ref, wp1_ref, wp2_ref,
             wpc_ref,
             wg0_ref, bg0_ref, wt0_ref, bt0_ref,
             wg1_ref, bg1_ref, wt1_ref, bt1_ref, out_ref):
  a = acc_ref[...]
  cnt = a[:, DC:DC + 1]
  scale = 1.0 / (0.001 + jnp.sqrt(cnt))
  ec = (a[:, :DC] * scale).astype(_bf16)

  def mm(lhs, w_ref):
    return jnp.dot(lhs.astype(_bf16), w_ref[...],
                   preferred_element_type=_f32)

  x = mm(e0_ref[...], wp0_ref)
  x = x + mm(e1_ref[...], wp1_ref)
  x = x + mm(e2_ref[...], wp2_ref)
  x = x + jnp.dot(ec, wpc_ref[...], preferred_element_type=_f32)
  for wg, bg, wt, bt in ((wg0_ref, bg0_ref, wt0_ref, bt0_ref),
                         (wg1_ref, bg1_ref, wt1_ref, bt1_ref)):
    g = jax.nn.sigmoid(mm(x, wg) + bg[...])
    t = jax.nn.relu(mm(x, wt) + bt[...])
    x = g * t + (1.0 - g) * x
  out_ref[...] = x


ROWB = 512
_N_BLK = B * LW // ROWB

_w_spec = pl.BlockSpec((H, H), lambda i: (0, 0))
_b_spec = pl.BlockSpec((1, H), lambda i: (0, 0))

_tc_call = pl.pallas_call(
    _tc_body,
    grid=(_N_BLK,),
    in_specs=[
        pl.BlockSpec((ROWB, 128), lambda i: (i, 0)),
        pl.BlockSpec((ROWB, 128), lambda i: (i, 0)),
        pl.BlockSpec((ROWB, 128), lambda i: (i, 0)),
        pl.BlockSpec((ROWB, DCP), lambda i: (i, 0)),
        pl.BlockSpec((128, H), lambda i: (0, 0)),
        pl.BlockSpec((128, H), lambda i: (0, 0)),
        pl.BlockSpec((128, H), lambda i: (0, 0)),
        pl.BlockSpec((DC, H), lambda i: (0, 0)),
        _w_spec, _b_spec, _w_spec, _b_spec,
        _w_spec, _b_spec, _w_spec, _b_spec,
    ],
    out_specs=pl.BlockSpec((ROWB, H), lambda i: (i, 0)),
    out_shape=jax.ShapeDtypeStruct((B * LW, H), _f32),
)


def kernel(w, c, p, word_vectors, char_table, W_proj,
           Wt0, bt0, Wg0, bg0, Wt1, bt1, Wg1, bg1):
  w3 = w.astype(_i32).reshape(NW, NWCHUNK, WCHUNK)
  c3 = c.astype(_i32).reshape(B, NS, CCHUNK)
  p3 = p.astype(_i32).reshape(B, NS, CCHUNK)
  ctab = jnp.concatenate(
      [char_table.astype(_f32),
       jnp.ones((257, 1), _f32),
       jnp.zeros((257, DCP - DC - 1), _f32)], axis=1)

  acc = _scc_call(c3, p3, ctab)

  wvp = jnp.pad(word_vectors.astype(_f32), ((0, 0), (0, DWP - DW)))
  e0, e1, e2 = _scw_call(w3, wvp, acc)

  wp_bf = W_proj.astype(_bf16)
  x = _tc_call(
      e0, e1, e2, acc.reshape(B * LW, DCP),
      wp_bf[0:128], wp_bf[128:256],
      jnp.pad(wp_bf[256:DW], ((0, 128 - (DW - 256)), (0, 0))), wp_bf[DW:],
      Wg0.astype(_bf16), bg0.reshape(1, H), Wt0.astype(_bf16),
      bt0.reshape(1, H),
      Wg1.astype(_bf16), bg1.reshape(1, H), Wt1.astype(_bf16),
      bt1.reshape(1, H))
  return x.reshape(B, LW, H)


# speedup vs baseline: 2.0905x; 2.0905x over previous
import functools

import jax
import jax.numpy as jnp
from jax import lax
from jax.experimental import pallas as pl
from jax.experimental.pallas import tpu as pltpu
from jax.experimental.pallas import tpu_sc as plsc

B, LW, LC = 64, 512, 2048
DW, DC, H = 300, 64, 128
DWP = 384
DCP = 128

NC, NS = 2, 16
NW = NC * NS

ROWS_PER_W = B * LW // NW
WCHUNK = 128
NWCHUNK = ROWS_PER_W // WCHUNK
NPIECE = DWP // 128
CCHUNK = 128
ZROWS = LW // NS

_f32 = jnp.float32
_bf16 = jnp.bfloat16
_i32 = jnp.int32

_MESH = plsc.VectorSubcoreMesh(core_axis_name="c", subcore_axis_name="s")


def _scc_body(c_hbm, p_hbm, ctab_hbm, acc_hbm,
              cbuf, pbuf, gbuf, zbuf, acc, sem):
  sid = lax.axis_index("s")
  cid = lax.axis_index("c")

  zseg = jnp.zeros((16,), _f32)

  def _zero_row(i, _):
    for q in range(DCP // 16):
      zbuf[i, pl.ds(q * 16, 16)] = zseg
    return 0

  lax.fori_loop(0, ZROWS, _zero_row, 0)

  myrows = pl.ds(sid * ZROWS, ZROWS)

  def _row(j, _):
    b = cid * (B // NC) + j
    pltpu.sync_copy(c_hbm.at[b, sid], cbuf)
    pltpu.sync_copy(p_hbm.at[b, sid], pbuf)

    for q in range(CCHUNK // 16):
      v = cbuf[pl.ds(q * 16, 16)]
      v = jnp.where(v == 32, 0, v)
      v = jnp.minimum(v, 256)
      cbuf[pl.ds(q * 16, 16)] = v

    pltpu.sync_copy(zbuf, acc.at[myrows])
    plsc.subcore_barrier()
    pltpu.async_copy(ctab_hbm.at[cbuf], gbuf, sem).wait()
    pltpu.sync_copy(gbuf, acc.at[pbuf], add=True)
    plsc.subcore_barrier()
    pltpu.sync_copy(acc.at[myrows], acc_hbm.at[b, myrows])
    plsc.subcore_barrier()
    return 0

  lax.fori_loop(0, B // NC, _row, 0)


_scc_call = functools.partial(
    pl.kernel,
    out_type=jax.ShapeDtypeStruct((B, LW, DCP), _f32),
    mesh=_MESH,
    scratch_types=[
        pltpu.VMEM((CCHUNK,), _i32),
        pltpu.VMEM((CCHUNK,), _i32),
        pltpu.VMEM((CCHUNK, DCP), _f32),
        pltpu.VMEM((ZROWS, DCP), _f32),
        pltpu.VMEM_SHARED((LW, DCP), _f32),
        pltpu.SemaphoreType.DMA,
    ],
)(_scc_body)


def _scw_body(w_hbm, table_hbm, ttail_hbm, dep_hbm,
              e0_hbm, e1_hbm, e2_hbm,
              widx, b0a, b1a, b2a, b0b, b1b, b2b, gsem, ssem):
  del dep_hbm
  sid = lax.axis_index("s")
  wid = sid * NC + lax.axis_index("c")

  pltpu.sync_copy(w_hbm.at[wid], widx)
  wbase = wid * ROWS_PER_W
  srcs = (table_hbm.at[:, pl.ds(0, 128)], table_hbm.at[:, pl.ds(128, 128)],
          ttail_hbm)
  outs = (e0_hbm, e1_hbm, e2_hbm)
  bufsets = ((b0a, b1a, b2a), (b0b, b1b, b2b))

  del bufsets, ssem
  for k in range(NWCHUNK):
    rows = pl.ds(wbase + k * WCHUNK, WCHUNK)
    hs = [pltpu.async_copy(srcs[i].at[widx.at[k]], (b0a, b1a, b2a)[i], gsem)
          for i in range(NPIECE)]
    for h in hs:
      h.wait()
    for i in range(NPIECE):
      pltpu.sync_copy((b0a, b1a, b2a)[i], outs[i].at[rows])


_scw_call = functools.partial(
    pl.kernel,
    out_type=(
        jax.ShapeDtypeStruct((B * LW, 128), _f32),
        jax.ShapeDtypeStruct((B * LW, 128), _f32),
        jax.ShapeDtypeStruct((B * LW, 128), _f32),
    ),
    mesh=_MESH,
    scratch_types=[
        pltpu.VMEM((NWCHUNK, WCHUNK), _i32),
        pltpu.VMEM((WCHUNK, 128), _f32),
        pltpu.VMEM((WCHUNK, 128), _f32),
        pltpu.VMEM((WCHUNK, 128), _f32),
        pltpu.VMEM((WCHUNK, 128), _f32),
        pltpu.VMEM((WCHUNK, 128), _f32),
        pltpu.VMEM((WCHUNK, 128), _f32),
        pltpu.SemaphoreType.DMA,
        pltpu.SemaphoreType.DMA,
    ],
)(_scw_body)


def _tc_body(e0_ref, e1_ref, e2_ref, acc_ref, wp0_ref, wp1_ref, wp2_ref,
             wpc_ref,
             wg0_ref, bg0_ref, wt0_ref, bt0_ref,
             wg1_ref, bg1_ref, wt1_ref, bt1_ref, out_ref):
  a = acc_ref[...]
  cnt = a[:, DC:DC + 1]
  scale = 1.0 / (0.001 + jnp.sqrt(cnt))
  ec = (a[:, :DC] * scale).astype(_bf16)

  def mm(lhs, w_ref):
    return jnp.dot(lhs.astype(_bf16), w_ref[...],
                   preferred_element_type=_f32)

  x = mm(e0_ref[...], wp0_ref)
  x = x + mm(e1_ref[...], wp1_ref)
  x = x + mm(e2_ref[...], wp2_ref)
  x = x + jnp.dot(ec, wpc_ref[...], preferred_element_type=_f32)
  for wg, bg, wt, bt in ((wg0_ref, bg0_ref, wt0_ref, bt0_ref),
                         (wg1_ref, bg1_ref, wt1_ref, bt1_ref)):
    g = jax.nn.sigmoid(mm(x, wg) + bg[...])
    t = jax.nn.relu(mm(x, wt) + bt[...])
    x = g * t + (1.0 - g) * x
  out_ref[...] = x


ROWB = 512
_N_BLK = B * LW // ROWB

_w_spec = pl.BlockSpec((H, H), lambda i: (0, 0))
_b_spec = pl.BlockSpec((1, H), lambda i: (0, 0))

_tc_call = pl.pallas_call(
    _tc_body,
    grid=(_N_BLK,),
    in_specs=[
        pl.BlockSpec((ROWB, 128), lambda i: (i, 0)),
        pl.BlockSpec((ROWB, 128), lambda i: (i, 0)),
        pl.BlockSpec((ROWB, 128), lambda i: (i, 0)),
        pl.BlockSpec((ROWB, DCP), lambda i: (i, 0)),
        pl.BlockSpec((128, H), lambda i: (0, 0)),
        pl.BlockSpec((128, H), lambda i: (0, 0)),
        pl.BlockSpec((128, H), lambda i: (0, 0)),
        pl.BlockSpec((DC, H), lambda i: (0, 0)),
        _w_spec, _b_spec, _w_spec, _b_spec,
        _w_spec, _b_spec, _w_spec, _b_spec,
    ],
    out_specs=pl.BlockSpec((ROWB, H), lambda i: (i, 0)),
    out_shape=jax.ShapeDtypeStruct((B * LW, H), _f32),
)


def kernel(w, c, p, word_vectors, char_table, W_proj,
           Wt0, bt0, Wg0, bg0, Wt1, bt1, Wg1, bg1):
  w3 = w.astype(_i32).reshape(NW, NWCHUNK, WCHUNK)
  c3 = c.astype(_i32).reshape(B, NS, CCHUNK)
  p3 = p.astype(_i32).reshape(B, NS, CCHUNK)
  ctab = jnp.concatenate(
      [char_table.astype(_f32),
       jnp.ones((257, 1), _f32),
       jnp.zeros((257, DCP - DC - 1), _f32)], axis=1)

  acc = _scc_call(c3, p3, ctab)

  wv = word_vectors.astype(_f32)
  ttail = jnp.pad(wv[:, 256:DW], ((0, 0), (0, 128 - (DW - 256))))
  e0, e1, e2 = _scw_call(w3, wv, ttail, acc)

  wp_bf = W_proj.astype(_bf16)
  x = _tc_call(
      e0, e1, e2, acc.reshape(B * LW, DCP),
      wp_bf[0:128], wp_bf[128:256],
      jnp.pad(wp_bf[256:DW], ((0, 128 - (DW - 256)), (0, 0))), wp_bf[DW:],
      Wg0.astype(_bf16), bg0.reshape(1, H), Wt0.astype(_bf16),
      bt0.reshape(1, H),
      Wg1.astype(_bf16), bg1.reshape(1, H), Wt1.astype(_bf16),
      bt1.reshape(1, H))
  return x.reshape(B, LW, H)


# speedup vs baseline: 2.1269x; 1.0174x over previous
import functools

import jax
import jax.numpy as jnp
from jax import lax
from jax.experimental import pallas as pl
from jax.experimental.pallas import tpu as pltpu
from jax.experimental.pallas import tpu_sc as plsc

B, LW, LC = 64, 512, 2048
DW, DC, H = 300, 64, 128
DWP = 384
DCP = 128

NC, NS = 2, 16
NW = NC * NS

ROWS_PER_W = B * LW // NW
WCHUNK = 128
NWCHUNK = ROWS_PER_W // WCHUNK
NPIECE = DWP // 128
CCHUNK = 128
ZROWS = LW // NS

_f32 = jnp.float32
_bf16 = jnp.bfloat16
_i32 = jnp.int32

_MESH = plsc.VectorSubcoreMesh(core_axis_name="c", subcore_axis_name="s")


NGBUF = 4
NCCHUNK = LC // CCHUNK
BPW = B // NW


HALF = NS // 2
NROUND = (B // NC) // HALF
MYCHUNK = NCCHUNK // 2


def _scc_body(c_hbm, p_hbm, ctab_hbm, acc_hbm,
              cbuf, pbuf, g0, g1, g2, g3, zbuf, acc_sh, sem):
  sid = lax.axis_index("s")
  cid = lax.axis_index("c")
  lane = lax.rem(sid, HALF)
  half = sid // HALF
  slab = acc_sh.at[lane]
  gbufs = (g0, g1, g2, g3)
  chunk0 = half * MYCHUNK
  row0 = half * (LW // 2)

  zseg = jnp.zeros((16,), _f32)

  def _zero_row(i, _):
    for q in range(DCP // 16):
      zbuf[i, pl.ds(q * 16, 16)] = zseg
    return 0

  lax.fori_loop(0, CCHUNK, _zero_row, 0)

  for j in range(NROUND):
    b = cid * (B // NC) + j * HALF + lane
    pltpu.sync_copy(c_hbm.at[b, pl.ds(chunk0, MYCHUNK)], cbuf)
    pltpu.sync_copy(p_hbm.at[b, pl.ds(chunk0, MYCHUNK)], pbuf)

    def _remap(i, _):
      for q in range(CCHUNK // 16):
        v = cbuf[i, pl.ds(q * 16, 16)]
        v = jnp.where(v == 32, 0, v)
        v = jnp.minimum(v, 256)
        cbuf[i, pl.ds(q * 16, 16)] = v
      return 0

    lax.fori_loop(0, MYCHUNK, _remap, 0)

    for q in range(LW // 2 // CCHUNK):
      pltpu.sync_copy(zbuf, slab.at[pl.ds(row0 + q * CCHUNK, CCHUNK)])
    plsc.subcore_barrier()

    for grp in range(MYCHUNK // NGBUF):
      hs = [pltpu.async_copy(ctab_hbm.at[cbuf.at[grp * NGBUF + i]],
                             gbufs[i], sem)
            for i in range(NGBUF)]
      for i in range(NGBUF):
        hs[i].wait()
        pltpu.sync_copy(gbufs[i], slab.at[pbuf.at[grp * NGBUF + i]],
                        add=True)
    plsc.subcore_barrier()

    pltpu.sync_copy(slab.at[pl.ds(row0, LW // 2)],
                    acc_hbm.at[b, pl.ds(row0, LW // 2)])
    plsc.subcore_barrier()


_scc_call = functools.partial(
    pl.kernel,
    out_type=jax.ShapeDtypeStruct((B, LW, DCP), _f32),
    mesh=_MESH,
    scratch_types=[
        pltpu.VMEM((MYCHUNK, CCHUNK), _i32),
        pltpu.VMEM((MYCHUNK, CCHUNK), _i32),
        pltpu.VMEM((CCHUNK, DCP), _f32),
        pltpu.VMEM((CCHUNK, DCP), _f32),
        pltpu.VMEM((CCHUNK, DCP), _f32),
        pltpu.VMEM((CCHUNK, DCP), _f32),
        pltpu.VMEM((CCHUNK, DCP), _f32),
        pltpu.VMEM_SHARED((HALF, LW, DCP), _f32),
        pltpu.SemaphoreType.DMA,
    ],
)(_scc_body)


def _scw_body(w_hbm, table_hbm, ttail_hbm, dep_hbm,
              e0_hbm, e1_hbm, e2_hbm,
              widx, b0a, b1a, b2a, b0b, b1b, b2b, gsem, ssem):
  del dep_hbm
  sid = lax.axis_index("s")
  wid = sid * NC + lax.axis_index("c")

  pltpu.sync_copy(w_hbm.at[wid], widx)
  wbase = wid * ROWS_PER_W
  srcs = (table_hbm.at[:, pl.ds(0, 128)], table_hbm.at[:, pl.ds(128, 128)],
          ttail_hbm)
  outs = (e0_hbm, e1_hbm, e2_hbm)
  bufsets = ((b0a, b1a, b2a), (b0b, b1b, b2b))

  del bufsets, ssem
  for k in range(NWCHUNK):
    rows = pl.ds(wbase + k * WCHUNK, WCHUNK)
    hs = [pltpu.async_copy(srcs[i].at[widx.at[k]], (b0a, b1a, b2a)[i], gsem)
          for i in range(NPIECE)]
    for h in hs:
      h.wait()
    for i in range(NPIECE):
      pltpu.sync_copy((b0a, b1a, b2a)[i], outs[i].at[rows])


_scw_call = functools.partial(
    pl.kernel,
    out_type=(
        jax.ShapeDtypeStruct((B * LW, 128), _f32),
        jax.ShapeDtypeStruct((B * LW, 128), _f32),
        jax.ShapeDtypeStruct((B * LW, 128), _f32),
    ),
    mesh=_MESH,
    scratch_types=[
        pltpu.VMEM((NWCHUNK, WCHUNK), _i32),
        pltpu.VMEM((WCHUNK, 128), _f32),
        pltpu.VMEM((WCHUNK, 128), _f32),
        pltpu.VMEM((WCHUNK, 128), _f32),
        pltpu.VMEM((WCHUNK, 128), _f32),
        pltpu.VMEM((WCHUNK, 128), _f32),
        pltpu.VMEM((WCHUNK, 128), _f32),
        pltpu.SemaphoreType.DMA,
        pltpu.SemaphoreType.DMA,
    ],
)(_scw_body)


def _tc_body(e0_ref, e1_ref, e2_ref, acc_ref, wp0_ref, wp1_ref, wp2_ref,
             wpc_ref,
             wg0_ref, bg0_ref, wt0_ref, bt0_ref,
             wg1_ref, bg1_ref, wt1_ref, bt1_ref, out_ref):
  a = acc_ref[...]
  cnt = a[:, DC:DC + 1]
  scale = 1.0 / (0.001 + jnp.sqrt(cnt))
  ec = (a[:, :DC] * scale).astype(_bf16)

  def mm(lhs, w_ref):
    return jnp.dot(lhs.astype(_bf16), w_ref[...],
                   preferred_element_type=_f32)

  x = mm(e0_ref[...], wp0_ref)
  x = x + mm(e1_ref[...], wp1_ref)
  x = x + mm(e2_ref[...], wp2_ref)
  x = x + jnp.dot(ec, wpc_ref[...], preferred_element_type=_f32)
  for wg, bg, wt, bt in ((wg0_ref, bg0_ref, wt0_ref, bt0_ref),
                         (wg1_ref, bg1_ref, wt1_ref, bt1_ref)):
    g = jax.nn.sigmoid(mm(x, wg) + bg[...])
    t = jax.nn.relu(mm(x, wt) + bt[...])
    x = g * t + (1.0 - g) * x
  out_ref[...] = x


ROWB = 512
_N_BLK = B * LW // ROWB

_w_spec = pl.BlockSpec((H, H), lambda i: (0, 0))
_b_spec = pl.BlockSpec((1, H), lambda i: (0, 0))

_tc_call = pl.pallas_call(
    _tc_body,
    grid=(_N_BLK,),
    in_specs=[
        pl.BlockSpec((ROWB, 128), lambda i: (i, 0)),
        pl.BlockSpec((ROWB, 128), lambda i: (i, 0)),
        pl.BlockSpec((ROWB, 128), lambda i: (i, 0)),
        pl.BlockSpec((ROWB, DCP), lambda i: (i, 0)),
        pl.BlockSpec((128, H), lambda i: (0, 0)),
        pl.BlockSpec((128, H), lambda i: (0, 0)),
        pl.BlockSpec((128, H), lambda i: (0, 0)),
        pl.BlockSpec((DC, H), lambda i: (0, 0)),
        _w_spec, _b_spec, _w_spec, _b_spec,
        _w_spec, _b_spec, _w_spec, _b_spec,
    ],
    out_specs=pl.BlockSpec((ROWB, H), lambda i: (i, 0)),
    out_shape=jax.ShapeDtypeStruct((B * LW, H), _f32),
)


def kernel(w, c, p, word_vectors, char_table, W_proj,
           Wt0, bt0, Wg0, bg0, Wt1, bt1, Wg1, bg1):
  w3 = w.astype(_i32).reshape(NW, NWCHUNK, WCHUNK)
  c3 = c.astype(_i32).reshape(B, NS, CCHUNK)
  p3 = p.astype(_i32).reshape(B, NS, CCHUNK)
  ctab = jnp.concatenate(
      [char_table.astype(_f32),
       jnp.ones((257, 1), _f32),
       jnp.zeros((257, DCP - DC - 1), _f32)], axis=1)

  acc = _scc_call(c3, p3, ctab)

  wv = word_vectors.astype(_f32)
  ttail = jnp.pad(wv[:, 256:DW], ((0, 0), (0, 128 - (DW - 256))))
  e0, e1, e2 = _scw_call(w3, wv, ttail, acc)

  wp_bf = W_proj.astype(_bf16)
  x = _tc_call(
      e0, e1, e2, acc.reshape(B * LW, DCP),
      wp_bf[0:128], wp_bf[128:256],
      jnp.pad(wp_bf[256:DW], ((0, 128 - (DW - 256)), (0, 0))), wp_bf[DW:],
      Wg0.astype(_bf16), bg0.reshape(1, H), Wt0.astype(_bf16),
      bt0.reshape(1, H),
      Wg1.astype(_bf16), bg1.reshape(1, H), Wt1.astype(_bf16),
      bt1.reshape(1, H))
  return x.reshape(B, LW, H)


# speedup vs baseline: 2.3138x; 1.0879x over previous
import functools

import jax
import jax.numpy as jnp
from jax import lax
from jax.experimental import pallas as pl
from jax.experimental.pallas import tpu as pltpu
from jax.experimental.pallas import tpu_sc as plsc

B, LW, LC = 64, 512, 2048
DW, DC, H = 300, 64, 128
DWP = 384
DCP = 128

NC, NS = 2, 16
NW = NC * NS

ROWS_PER_W = B * LW // NW
WCHUNK = 128
NWCHUNK = ROWS_PER_W // WCHUNK
NPIECE = DWP // 128
CCHUNK = 128
ZROWS = LW // NS

_f32 = jnp.float32
_bf16 = jnp.bfloat16
_i32 = jnp.int32

_MESH = plsc.VectorSubcoreMesh(core_axis_name="c", subcore_axis_name="s")


NGBUF = 4
NCCHUNK = LC // CCHUNK
BPW = B // NW


HALF = NS // 2
NROUND = (B // NC) // HALF
MYCHUNK = NCCHUNK // 2


def _scc_body(c_hbm, p_hbm, ctab_hbm, acc_hbm,
              cbuf, pbuf, g0, g1, g2, g3, zbuf, acc_sh, gsem, ssem):
  sid = lax.axis_index("s")
  cid = lax.axis_index("c")
  lane = lax.rem(sid, HALF)
  half = sid // HALF
  slab = acc_sh.at[lane]
  gbufs = (g0, g1, g2, g3)
  chunk0 = half * MYCHUNK
  row0 = half * (LW // 2)

  zseg = jnp.zeros((16,), _f32)

  def _zero_row(i, _):
    for q in range(DCP // 16):
      zbuf[i, pl.ds(q * 16, 16)] = zseg
    return 0

  lax.fori_loop(0, CCHUNK, _zero_row, 0)

  for j in range(NROUND):
    b = cid * (B // NC) + j * HALF + lane
    pltpu.sync_copy(c_hbm.at[b, pl.ds(chunk0, MYCHUNK)], cbuf.at[j])
    pltpu.sync_copy(p_hbm.at[b, pl.ds(chunk0, MYCHUNK)], pbuf.at[j])

  def _remap(i, _):
    for q in range(CCHUNK // 16):
      v = cbuf[i // MYCHUNK, i % MYCHUNK, pl.ds(q * 16, 16)]
      v = jnp.where(v == 32, 0, v)
      v = jnp.minimum(v, 256)
      cbuf[i // MYCHUNK, i % MYCHUNK, pl.ds(q * 16, 16)] = v
    return 0

  lax.fori_loop(0, NROUND * MYCHUNK, _remap, 0)

  for q in range(LW // 2 // CCHUNK):
    pltpu.sync_copy(zbuf, slab.at[pl.ds(row0 + q * CCHUNK, CCHUNK)])
  plsc.subcore_barrier()

  for j in range(NROUND):
    b = cid * (B // NC) + j * HALF + lane
    for grp in range(MYCHUNK // NGBUF):
      hs = [pltpu.async_copy(
          ctab_hbm.at[cbuf.at[j, grp * NGBUF + i]],
          gbufs[i], gsem)
            for i in range(NGBUF)]
      ss = []
      for i in range(NGBUF):
        hs[i].wait()
        ss.append(pltpu.async_copy(
            gbufs[i], slab.at[pbuf.at[j, grp * NGBUF + i]],
            ssem, add=True))
      for s in ss:
        s.wait()
    plsc.subcore_barrier()

    pltpu.sync_copy(slab.at[pl.ds(row0, LW // 2)],
                    acc_hbm.at[b, pl.ds(row0, LW // 2)])
    if j + 1 < NROUND:
      for q in range(LW // 2 // CCHUNK):
        pltpu.sync_copy(zbuf, slab.at[pl.ds(row0 + q * CCHUNK, CCHUNK)])
    plsc.subcore_barrier()


_scc_call = functools.partial(
    pl.kernel,
    out_type=jax.ShapeDtypeStruct((B, LW, DCP), _f32),
    mesh=_MESH,
    scratch_types=[
        pltpu.VMEM((NROUND, MYCHUNK, CCHUNK), _i32),
        pltpu.VMEM((NROUND, MYCHUNK, CCHUNK), _i32),
        pltpu.VMEM((CCHUNK, DCP), _f32),
        pltpu.VMEM((CCHUNK, DCP), _f32),
        pltpu.VMEM((CCHUNK, DCP), _f32),
        pltpu.VMEM((CCHUNK, DCP), _f32),
        pltpu.VMEM((CCHUNK, DCP), _f32),
        pltpu.VMEM_SHARED((HALF, LW, DCP), _f32),
        pltpu.SemaphoreType.DMA,
        pltpu.SemaphoreType.DMA,
    ],
)(_scc_body)


def _scw_body(w_hbm, table_hbm, ttail_hbm, dep_hbm,
              e0_hbm, e1_hbm, e2_hbm,
              widx, b0a, b1a, b2a, b0b, b1b, b2b, gsem, ssem):
  del dep_hbm
  sid = lax.axis_index("s")
  wid = sid * NC + lax.axis_index("c")

  pltpu.sync_copy(w_hbm.at[wid], widx)
  wbase = wid * ROWS_PER_W
  srcs = (table_hbm.at[:, pl.ds(0, 128)], table_hbm.at[:, pl.ds(128, 128)],
          ttail_hbm)
  outs = (e0_hbm, e1_hbm, e2_hbm)
  bufsets = ((b0a, b1a, b2a), (b0b, b1b, b2b))

  del bufsets, ssem
  for k in range(NWCHUNK):
    rows = pl.ds(wbase + k * WCHUNK, WCHUNK)
    hs = [pltpu.async_copy(srcs[i].at[widx.at[k]], (b0a, b1a, b2a)[i], gsem)
          for i in range(NPIECE)]
    for h in hs:
      h.wait()
    for i in range(NPIECE):
      pltpu.sync_copy((b0a, b1a, b2a)[i], outs[i].at[rows])


_scw_call = functools.partial(
    pl.kernel,
    out_type=(
        jax.ShapeDtypeStruct((B * LW, 128), _f32),
        jax.ShapeDtypeStruct((B * LW, 128), _f32),
        jax.ShapeDtypeStruct((B * LW, 128), _f32),
    ),
    mesh=_MESH,
    scratch_types=[
        pltpu.VMEM((NWCHUNK, WCHUNK), _i32),
        pltpu.VMEM((WCHUNK, 128), _f32),
        pltpu.VMEM((WCHUNK, 128), _f32),
        pltpu.VMEM((WCHUNK, 128), _f32),
        pltpu.VMEM((WCHUNK, 128), _f32),
        pltpu.VMEM((WCHUNK, 128), _f32),
        pltpu.VMEM((WCHUNK, 128), _f32),
        pltpu.SemaphoreType.DMA,
        pltpu.SemaphoreType.DMA,
    ],
)(_scw_body)


def _tc_body(e0_ref, e1_ref, e2_ref, acc_ref, wp0_ref, wp1_ref, wp2_ref,
             wpc_ref,
             wg0_ref, bg0_ref, wt0_ref, bt0_ref,
             wg1_ref, bg1_ref, wt1_ref, bt1_ref, out_ref):
  a = acc_ref[...]
  cnt = a[:, DC:DC + 1]
  scale = 1.0 / (0.001 + jnp.sqrt(cnt))
  ec = (a[:, :DC] * scale).astype(_bf16)

  def mm(lhs, w_ref):
    return jnp.dot(lhs.astype(_bf16), w_ref[...],
                   preferred_element_type=_f32)

  x = mm(e0_ref[...], wp0_ref)
  x = x + mm(e1_ref[...], wp1_ref)
  x = x + mm(e2_ref[...], wp2_ref)
  x = x + jnp.dot(ec, wpc_ref[...], preferred_element_type=_f32)
  for wg, bg, wt, bt in ((wg0_ref, bg0_ref, wt0_ref, bt0_ref),
                         (wg1_ref, bg1_ref, wt1_ref, bt1_ref)):
    g = jax.nn.sigmoid(mm(x, wg) + bg[...])
    t = jax.nn.relu(mm(x, wt) + bt[...])
    x = g * t + (1.0 - g) * x
  out_ref[...] = x


ROWB = 2048
_N_BLK = B * LW // ROWB

_w_spec = pl.BlockSpec((H, H), lambda i: (0, 0))
_b_spec = pl.BlockSpec((1, H), lambda i: (0, 0))

_tc_call = pl.pallas_call(
    _tc_body,
    grid=(_N_BLK,),
    in_specs=[
        pl.BlockSpec((ROWB, 128), lambda i: (i, 0)),
        pl.BlockSpec((ROWB, 128), lambda i: (i, 0)),
        pl.BlockSpec((ROWB, 128), lambda i: (i, 0)),
        pl.BlockSpec((ROWB, DCP), lambda i: (i, 0)),
        pl.BlockSpec((128, H), lambda i: (0, 0)),
        pl.BlockSpec((128, H), lambda i: (0, 0)),
        pl.BlockSpec((128, H), lambda i: (0, 0)),
        pl.BlockSpec((DC, H), lambda i: (0, 0)),
        _w_spec, _b_spec, _w_spec, _b_spec,
        _w_spec, _b_spec, _w_spec, _b_spec,
    ],
    out_specs=pl.BlockSpec((ROWB, H), lambda i: (i, 0)),
    out_shape=jax.ShapeDtypeStruct((B * LW, H), _f32),
)


def kernel(w, c, p, word_vectors, char_table, W_proj,
           Wt0, bt0, Wg0, bg0, Wt1, bt1, Wg1, bg1):
  w3 = w.astype(_i32).reshape(NW, NWCHUNK, WCHUNK)
  c3 = c.astype(_i32).reshape(B, NS, CCHUNK)
  p3 = p.astype(_i32).reshape(B, NS, CCHUNK)
  ctab = jnp.concatenate(
      [char_table.astype(_f32),
       jnp.ones((257, 1), _f32),
       jnp.zeros((257, DCP - DC - 1), _f32)], axis=1)

  acc = _scc_call(c3, p3, ctab)

  wv = word_vectors.astype(_f32)
  ttail = jnp.pad(wv[:, 256:DW], ((0, 0), (0, 128 - (DW - 256))))
  e0, e1, e2 = _scw_call(w3, wv, ttail, acc)

  wp_bf = W_proj.astype(_bf16)
  x = _tc_call(
      e0, e1, e2, acc.reshape(B * LW, DCP),
      wp_bf[0:128], wp_bf[128:256],
      jnp.pad(wp_bf[256:DW], ((0, 128 - (DW - 256)), (0, 0))), wp_bf[DW:],
      Wg0.astype(_bf16), bg0.reshape(1, H), Wt0.astype(_bf16),
      bt0.reshape(1, H),
      Wg1.astype(_bf16), bg1.reshape(1, H), Wt1.astype(_bf16),
      bt1.reshape(1, H))
  return x.reshape(B, LW, H)


# speedup vs baseline: 2.8327x; 1.2243x over previous
import functools

import jax
import jax.numpy as jnp
from jax import lax
from jax.experimental import pallas as pl
from jax.experimental.pallas import tpu as pltpu
from jax.experimental.pallas import tpu_sc as plsc

B, LW, LC = 64, 512, 2048
DW, DC, H = 300, 64, 128
DWP = 384
DCP = 128

NC, NS = 2, 16
NW = NC * NS

ROWS_PER_W = B * LW // NW
WCHUNK = 128
NWCHUNK = ROWS_PER_W // WCHUNK
NPIECE = DWP // 128
CCHUNK = 128
ZROWS = LW // NS

_f32 = jnp.float32
_bf16 = jnp.bfloat16
_i32 = jnp.int32

_MESH = plsc.VectorSubcoreMesh(core_axis_name="c", subcore_axis_name="s")


NGBUF = 4
NCCHUNK = LC // CCHUNK
BPW = B // NW


HALF = NS // 2
NROUND = (B // NC) // HALF
MYCHUNK = NCCHUNK // 2


def _scc_body(c_hbm, p_hbm, ctab_hbm, acc_hbm,
              cbuf, pbuf, g0, g1, g2, g3, zbuf, acc_sh, ctab_sh,
              gsem, ssem):
  sid = lax.axis_index("s")
  cid = lax.axis_index("c")
  lane = lax.rem(sid, HALF)
  half = sid // HALF
  slab = acc_sh.at[lane]
  gbufs = (g0, g1, g2, g3)
  chunk0 = half * MYCHUNK
  row0 = half * (LW // 2)

  zseg = jnp.zeros((16,), _f32)

  def _zero_row(i, _):
    for q in range(DCP // 16):
      zbuf[i, pl.ds(q * 16, 16)] = zseg
    return 0

  lax.fori_loop(0, CCHUNK, _zero_row, 0)

  for j in range(NROUND):
    b = cid * (B // NC) + j * HALF + lane
    pltpu.sync_copy(c_hbm.at[b, pl.ds(chunk0, MYCHUNK)], cbuf.at[j])
    pltpu.sync_copy(p_hbm.at[b, pl.ds(chunk0, MYCHUNK)], pbuf.at[j])

  def _remap(i, _):
    for q in range(CCHUNK // 16):
      v = cbuf[i // MYCHUNK, i % MYCHUNK, pl.ds(q * 16, 16)]
      v = jnp.where(v == 32, 0, v)
      v = jnp.minimum(v, 256)
      cbuf[i // MYCHUNK, i % MYCHUNK, pl.ds(q * 16, 16)] = v
    return 0

  lax.fori_loop(0, NROUND * MYCHUNK, _remap, 0)

  @pl.when(jnp.logical_and(sid == 0, True))
  def _():
    pltpu.sync_copy(ctab_hbm, ctab_sh)

  for q in range(LW // 2 // CCHUNK):
    pltpu.sync_copy(zbuf, slab.at[pl.ds(row0 + q * CCHUNK, CCHUNK)])
  plsc.subcore_barrier()

  for j in range(NROUND):
    b = cid * (B // NC) + j * HALF + lane
    for grp in range(MYCHUNK // NGBUF):
      hs = [pltpu.async_copy(
          ctab_sh.at[cbuf.at[j, grp * NGBUF + i]],
          gbufs[i], gsem)
            for i in range(NGBUF)]
      ss = []
      for i in range(NGBUF):
        hs[i].wait()
        ss.append(pltpu.async_copy(
            gbufs[i], slab.at[pbuf.at[j, grp * NGBUF + i]],
            ssem, add=True))
      for s in ss:
        s.wait()
    plsc.subcore_barrier()

    pltpu.sync_copy(slab.at[pl.ds(row0, LW // 2)],
                    acc_hbm.at[b, pl.ds(row0, LW // 2)])
    if j + 1 < NROUND:
      for q in range(LW // 2 // CCHUNK):
        pltpu.sync_copy(zbuf, slab.at[pl.ds(row0 + q * CCHUNK, CCHUNK)])
    plsc.subcore_barrier()


_scc_call = functools.partial(
    pl.kernel,
    out_type=jax.ShapeDtypeStruct((B, LW, DCP), _f32),
    mesh=_MESH,
    scratch_types=[
        pltpu.VMEM((NROUND, MYCHUNK, CCHUNK), _i32),
        pltpu.VMEM((NROUND, MYCHUNK, CCHUNK), _i32),
        pltpu.VMEM((CCHUNK, DCP), _f32),
        pltpu.VMEM((CCHUNK, DCP), _f32),
        pltpu.VMEM((CCHUNK, DCP), _f32),
        pltpu.VMEM((CCHUNK, DCP), _f32),
        pltpu.VMEM((CCHUNK, DCP), _f32),
        pltpu.VMEM_SHARED((HALF, LW, DCP), _f32),
        pltpu.VMEM_SHARED((257, DCP), _f32),
        pltpu.SemaphoreType.DMA,
        pltpu.SemaphoreType.DMA,
    ],
)(_scc_body)


def _scw_body(w_hbm, table_hbm, ttail_hbm, dep_hbm,
              e0_hbm, e1_hbm, e2_hbm,
              widx, b0a, b1a, b2a, b0b, b1b, b2b, gsem, ssem):
  del dep_hbm
  sid = lax.axis_index("s")
  wid = sid * NC + lax.axis_index("c")

  pltpu.sync_copy(w_hbm.at[wid], widx)
  wbase = wid * ROWS_PER_W
  srcs = (table_hbm.at[:, pl.ds(0, 128)], table_hbm.at[:, pl.ds(128, 128)],
          ttail_hbm)
  outs = (e0_hbm, e1_hbm, e2_hbm)
  bufsets = ((b0a, b1a, b2a), (b0b, b1b, b2b))

  del bufsets, ssem
  for k in range(NWCHUNK):
    rows = pl.ds(wbase + k * WCHUNK, WCHUNK)
    hs = [pltpu.async_copy(srcs[i].at[widx.at[k]], (b0a, b1a, b2a)[i], gsem)
          for i in range(NPIECE)]
    for h in hs:
      h.wait()
    for i in range(NPIECE):
      pltpu.sync_copy((b0a, b1a, b2a)[i], outs[i].at[rows])


_scw_call = functools.partial(
    pl.kernel,
    out_type=(
        jax.ShapeDtypeStruct((B * LW, 128), _f32),
        jax.ShapeDtypeStruct((B * LW, 128), _f32),
        jax.ShapeDtypeStruct((B * LW, 128), _f32),
    ),
    mesh=_MESH,
    scratch_types=[
        pltpu.VMEM((NWCHUNK, WCHUNK), _i32),
        pltpu.VMEM((WCHUNK, 128), _f32),
        pltpu.VMEM((WCHUNK, 128), _f32),
        pltpu.VMEM((WCHUNK, 128), _f32),
        pltpu.VMEM((WCHUNK, 128), _f32),
        pltpu.VMEM((WCHUNK, 128), _f32),
        pltpu.VMEM((WCHUNK, 128), _f32),
        pltpu.SemaphoreType.DMA,
        pltpu.SemaphoreType.DMA,
    ],
)(_scw_body)


def _tc_body(e0_ref, e1_ref, e2_ref, acc_ref, wp0_ref, wp1_ref, wp2_ref,
             wpc_ref,
             wg0_ref, bg0_ref, wt0_ref, bt0_ref,
             wg1_ref, bg1_ref, wt1_ref, bt1_ref, out_ref):
  a = acc_ref[...]
  cnt = a[:, DC:DC + 1]
  scale = 1.0 / (0.001 + jnp.sqrt(cnt))
  ec = (a[:, :DC] * scale).astype(_bf16)

  def mm(lhs, w_ref):
    return jnp.dot(lhs.astype(_bf16), w_ref[...],
                   preferred_element_type=_f32)

  x = mm(e0_ref[...], wp0_ref)
  x = x + mm(e1_ref[...], wp1_ref)
  x = x + mm(e2_ref[...], wp2_ref)
  x = x + jnp.dot(ec, wpc_ref[...], preferred_element_type=_f32)
  for wg, bg, wt, bt in ((wg0_ref, bg0_ref, wt0_ref, bt0_ref),
                         (wg1_ref, bg1_ref, wt1_ref, bt1_ref)):
    g = jax.nn.sigmoid(mm(x, wg) + bg[...])
    t = jax.nn.relu(mm(x, wt) + bt[...])
    x = g * t + (1.0 - g) * x
  out_ref[...] = x


ROWB = 2048
_N_BLK = B * LW // ROWB

_w_spec = pl.BlockSpec((H, H), lambda i: (0, 0))
_b_spec = pl.BlockSpec((1, H), lambda i: (0, 0))

_tc_call = pl.pallas_call(
    _tc_body,
    grid=(_N_BLK,),
    in_specs=[
        pl.BlockSpec((ROWB, 128), lambda i: (i, 0)),
        pl.BlockSpec((ROWB, 128), lambda i: (i, 0)),
        pl.BlockSpec((ROWB, 128), lambda i: (i, 0)),
        pl.BlockSpec((ROWB, DCP), lambda i: (i, 0)),
        pl.BlockSpec((128, H), lambda i: (0, 0)),
        pl.BlockSpec((128, H), lambda i: (0, 0)),
        pl.BlockSpec((128, H), lambda i: (0, 0)),
        pl.BlockSpec((DC, H), lambda i: (0, 0)),
        _w_spec, _b_spec, _w_spec, _b_spec,
        _w_spec, _b_spec, _w_spec, _b_spec,
    ],
    out_specs=pl.BlockSpec((ROWB, H), lambda i: (i, 0)),
    out_shape=jax.ShapeDtypeStruct((B * LW, H), _f32),
)


def kernel(w, c, p, word_vectors, char_table, W_proj,
           Wt0, bt0, Wg0, bg0, Wt1, bt1, Wg1, bg1):
  w3 = w.astype(_i32).reshape(NW, NWCHUNK, WCHUNK)
  c3 = c.astype(_i32).reshape(B, NS, CCHUNK)
  p3 = p.astype(_i32).reshape(B, NS, CCHUNK)
  ctab = jnp.concatenate(
      [char_table.astype(_f32),
       jnp.ones((257, 1), _f32),
       jnp.zeros((257, DCP - DC - 1), _f32)], axis=1)

  acc = _scc_call(c3, p3, ctab)

  wv = word_vectors.astype(_f32)
  ttail = jnp.pad(wv[:, 256:DW], ((0, 0), (0, 128 - (DW - 256))))
  e0, e1, e2 = _scw_call(w3, wv, ttail, acc)

  wp_bf = W_proj.astype(_bf16)
  x = _tc_call(
      e0, e1, e2, acc.reshape(B * LW, DCP),
      wp_bf[0:128], wp_bf[128:256],
      jnp.pad(wp_bf[256:DW], ((0, 128 - (DW - 256)), (0, 0))), wp_bf[DW:],
      Wg0.astype(_bf16), bg0.reshape(1, H), Wt0.astype(_bf16),
      bt0.reshape(1, H),
      Wg1.astype(_bf16), bg1.reshape(1, H), Wt1.astype(_bf16),
      bt1.reshape(1, H))
  return x.reshape(B, LW, H)
